# baseline (speedup 1.0000x reference)
import functools

import jax
import jax.numpy as jnp
from jax import lax
from jax.experimental import pallas as pl
from jax.experimental.pallas import tpu as pltpu
from jax.experimental.pallas import tpu_sc as plsc

_NUM_NODE = 5000
_EMBED = 128
_NUM_CLS = 20
_BZ, _SEQ, _NBR = 1024, 50, 8
_PAIRS = _SEQ * _NBR
_SEQ_PAD = 56
_IDXW = _PAIRS + _PAIRS + _SEQ_PAD
_LANES = 16
_NVEC = _EMBED // _LANES

_info = plsc.get_sparse_core_info()
_NC, _NS = _info.num_cores, _info.num_subcores
_NW = _NC * _NS
_B_PER_W = _BZ // _NW

_CHUNKS = ((0, 128), (128, 128), (256, 128), (384, 16))


@functools.partial(
    pl.kernel,
    out_type=jax.ShapeDtypeStruct((_BZ, _EMBED), jnp.float32),
    mesh=plsc.VectorSubcoreMesh(core_axis_name="c", subcore_axis_name="s"),
    scratch_types=[
        pltpu.VMEM((_IDXW,), jnp.int32),
        pltpu.VMEM((_IDXW,), jnp.int32),
        pltpu.VMEM((_PAIRS, _EMBED), jnp.float32),
        pltpu.VMEM((_PAIRS, _EMBED), jnp.float32),
        pltpu.VMEM((_PAIRS + _LANES,), jnp.float32),
        pltpu.VMEM((_PAIRS + _LANES,), jnp.float32),
        pltpu.VMEM((_SEQ_PAD, _EMBED), jnp.float32),
        pltpu.VMEM((_SEQ_PAD, _EMBED), jnp.float32),
        pltpu.VMEM((_SEQ_PAD + _LANES,), jnp.float32),
        pltpu.VMEM((_SEQ_PAD + _LANES,), jnp.float32),
        pltpu.VMEM((_B_PER_W, _EMBED), jnp.float32),
        pltpu.SemaphoreType.DMA,
        pltpu.SemaphoreType.DMA,
    ],
)
def _sc_pool(emb_hbm, ew_hbm, nw_hbm, idx_hbm, out_hbm,
             idx0, idx1, rows0, rows1, ewv0, ewv1, rn0, rn1, nn0, nn1,
             hblk, sem0, sem1):
    wid = lax.axis_index("s") * _NC + lax.axis_index("c")
    b0 = wid * _B_PER_W
    slots = ((idx0, rows0, ewv0, rn0, nn0, sem0),
             (idx1, rows1, ewv1, rn1, nn1, sem1))

    def copies(slot):
        idx, rows, ewv, rn, nn, sem = slots[slot]
        cs = []
        for off, sz in _CHUNKS:
            cs.append(pltpu.make_async_copy(
                emb_hbm.at[idx.at[pl.ds(off, sz)]],
                rows.at[pl.ds(off, sz)], sem))
            cs.append(pltpu.make_async_copy(
                ew_hbm.at[idx.at[pl.ds(_PAIRS + off, sz)]],
                ewv.at[pl.ds(off, sz)], sem))
        cs.append(pltpu.make_async_copy(
            emb_hbm.at[idx.at[pl.ds(2 * _PAIRS, _SEQ_PAD)]], rn, sem))
        cs.append(pltpu.make_async_copy(
            nw_hbm.at[idx.at[pl.ds(2 * _PAIRS, _SEQ_PAD)]],
            nn.at[pl.ds(0, _SEQ_PAD)], sem))
        return cs

    def fetch(slot, lb):
        pltpu.sync_copy(idx_hbm.at[b0 + lb], slots[slot][0])
        for c in copies(slot):
            c.start()

    def compute(slot, lb):
        _, rows, ewv, rn, nn, _ = slots[slot]
        for c in copies(slot):
            c.wait()

        def s_body(s, acc):
            base = s * _NBR
            wv = ewv[pl.ds(base, _LANES)]
            m = [None] * _NVEC
            for n in range(_NBR):
                wb = jnp.full((_LANES,), wv[n], jnp.float32)
                for e in range(_NVEC):
                    v = rows[base + n, pl.ds(e * _LANES, _LANES)] * wb
                    m[e] = v if n == 0 else jnp.maximum(m[e], v)
            nb = jnp.full((_LANES,), nn[pl.ds(s, _LANES)][0], jnp.float32)
            ob = 1.0 - nb
            return tuple(
                acc[e] + ob * m[e] + nb * rn[s, pl.ds(e * _LANES, _LANES)]
                for e in range(_NVEC))

        acc0 = tuple(jnp.zeros((_LANES,), jnp.float32) for _ in range(_NVEC))
        acc = lax.fori_loop(0, _SEQ, s_body, acc0)
        for e in range(_NVEC):
            hblk[lb, pl.ds(e * _LANES, _LANES)] = acc[e]

    fetch(0, 0)

    def step_body(step, carry):
        lb0 = 2 * step
        fetch(1, lb0 + 1)
        compute(0, lb0)

        @pl.when(step < _B_PER_W // 2 - 1)
        def _():
            fetch(0, lb0 + 2)

        compute(1, lb0 + 1)
        return carry

    lax.fori_loop(0, _B_PER_W // 2, step_body, 0)
    pltpu.sync_copy(hblk, out_hbm.at[pl.ds(b0, _B_PER_W)])


def _fc_body(h_ref, w_ref, b_ref, o_ref):
    z = jnp.dot(h_ref[...], w_ref[...], preferred_element_type=jnp.float32)
    z = jnp.maximum(z + b_ref[...], 0.0)
    mx = jnp.max(z, axis=1, keepdims=True)
    ez = jnp.exp(z - mx)
    lse = jnp.log(jnp.sum(ez, axis=1, keepdims=True)) + mx
    o_ref[...] = z - lse


def kernel(X, NX, EW, node_emb, edge_w, node_w, fc_W, fc_b):
    idx_all = jnp.concatenate([
        NX.reshape(_BZ, _PAIRS).astype(jnp.int32),
        EW.reshape(_BZ, _PAIRS).astype(jnp.int32),
        jnp.pad(X.astype(jnp.int32), ((0, 0), (0, _SEQ_PAD - _SEQ))),
    ], axis=1)
    ew_flat = jnp.reshape(jnp.transpose(edge_w, (1, 0)), (-1,))
    h = _sc_pool(node_emb, ew_flat, node_w.reshape(-1), idx_all)
    return pl.pallas_call(
        _fc_body,
        out_shape=jax.ShapeDtypeStruct((_BZ, _NUM_CLS), jnp.float32),
    )(h, fc_W, fc_b.reshape(1, _NUM_CLS))
